# baseline (speedup 1.0000x reference)
import functools

import jax
import jax.numpy as jnp
from jax import lax
from jax.experimental import pallas as pl
from jax.experimental.pallas import tpu as pltpu
from jax.experimental.pallas import tpu_sc as plsc

N = 10000
D = 128
E = 320000

NC = 2
NS = 16
NW = NC * NS
L = 16

K = 80
NB = 4
RB = 4
HK = K // 2
EPW = E // NW
NCHUNK = EPW // K
NACC = 10112
ZPT = NACC // NS
DPT = 624


def _sc_body(eidx_hbm, w_hbm, ann_hbm, part_hbm,
             col_v, row_v, w_v, srow_v, rows_v, acc, gsem, gsem2,
             isem, ssem):
    cid = lax.axis_index("c")
    sid = lax.axis_index("s")
    wid = sid * NC + cid
    ebase = wid * EPW

    def idx_descs(c, q):
        base = pl.multiple_of(ebase + c * K, K)
        return (
            pltpu.make_async_copy(eidx_hbm.at[pl.ds(base, K)], row_v.at[q],
                                  isem.at[q]),
            pltpu.make_async_copy(eidx_hbm.at[pl.ds(E + base, K)],
                                  col_v.at[q], isem.at[q]),
            pltpu.make_async_copy(w_hbm.at[pl.ds(base, K)], w_v.at[q],
                                  isem.at[q]),
        )

    def gather_descs(q, p):
        return (
            pltpu.make_async_copy(ann_hbm.at[col_v.at[q, pl.ds(0, HK)]],
                                  rows_v.at[p, pl.ds(0, HK)], gsem.at[p]),
            pltpu.make_async_copy(ann_hbm.at[col_v.at[q, pl.ds(HK, HK)]],
                                  rows_v.at[p, pl.ds(HK, HK)], gsem2.at[p]),
        )

    def gather_start(q, p):
        for d in gather_descs(q, p):
            d.start()

    def gather_wait(q, p):
        for d in gather_descs(q, p):
            d.wait()

    def scatter_start(p):
        pltpu.async_copy(rows_v.at[p], acc.at[srow_v.at[p]], ssem.at[p],
                         add=True)

    def scatter_wait(p):
        pltpu.make_async_copy(rows_v.at[p], acc.at[srow_v.at[p]],
                              ssem.at[p]).wait()

    def mul_chunk(q, p):
        def mul_body(g, _):
            wv = w_v[q, pl.ds(g * L, L)]
            for l in range(L):
                e = g * L + l
                ws = jnp.full((L,), wv[l], jnp.float32)
                for j in range(D // L):
                    rows_v[p, e, pl.ds(j * L, L)] = (
                        rows_v[p, e, pl.ds(j * L, L)] * ws)
            return 0
        lax.fori_loop(0, K // L, mul_body, 0)
        for j in range(K // L):
            srow_v[p, pl.ds(j * L, L)] = row_v[q, pl.ds(j * L, L)]

    def zero_rows(e, _):
        for j in range(D // L):
            rows_v[0, e, pl.ds(j * L, L)] = jnp.zeros((L,), jnp.float32)
        return 0
    lax.fori_loop(0, K, zero_rows, 0)
    zbase = sid * ZPT
    for i in range(ZPT // K):
        pltpu.sync_copy(rows_v.at[0, pl.ds(0, K)],
                        acc.at[pl.ds(zbase + i * K, K)])
    pltpu.sync_copy(rows_v.at[0, pl.ds(0, ZPT - (ZPT // K) * K)],
                    acc.at[pl.ds(zbase + (ZPT // K) * K,
                                 ZPT - (ZPT // K) * K)])
    plsc.subcore_barrier()

    for c in range(NB):
        for d in idx_descs(c, c):
            d.start()
    for c in range(2):
        for d in idx_descs(c, c):
            d.wait()
        gather_start(c, c)

    def chunk_body(c4, _):
        for k in range(NB):
            kn2 = (k + 2) % NB
            c = c4 * NB + k
            gather_wait(k, k)

            @pl.when(c >= 2)
            def _():
                scatter_wait(kn2)

            @pl.when(c < NCHUNK - 2)
            def _():
                for d in idx_descs(c + 2, kn2):
                    d.wait()
                gather_start(kn2, kn2)

            mul_chunk(k, k)
            scatter_start(k)

            @pl.when(c < NCHUNK - NB)
            def _():
                for d in idx_descs(c + NB, k):
                    d.start()
        return 0
    lax.fori_loop(0, (NCHUNK - 1) // NB, chunk_body, 0)

    kf = (NCHUNK - 1) % NB
    gather_wait(kf, kf)
    scatter_wait((NCHUNK - 3) % RB)
    mul_chunk(kf, kf)
    scatter_start(kf)
    scatter_wait((NCHUNK - 2) % RB)
    scatter_wait(kf)
    plsc.subcore_barrier()

    dbase = DPT * sid + 8 * jnp.minimum(sid, 2)
    pltpu.sync_copy(acc.at[pl.ds(dbase, DPT)],
                    part_hbm.at[pl.ds(cid * N + dbase, DPT)])

    @pl.when(sid < 2)
    def _():
        pltpu.sync_copy(acc.at[pl.ds(dbase + DPT, 8)],
                        part_hbm.at[pl.ds(cid * N + dbase + DPT, 8)])


@functools.cache
def _sc_call():
    return pl.kernel(
        _sc_body,
        out_type=jax.ShapeDtypeStruct((NC * N, D), jnp.float32),
        mesh=plsc.VectorSubcoreMesh(core_axis_name="c", subcore_axis_name="s",
                                    num_cores=NC, num_subcores=NS),
        scratch_types=[
            pltpu.VMEM((NB, K), jnp.int32),
            pltpu.VMEM((NB, K), jnp.int32),
            pltpu.VMEM((NB, K), jnp.float32),
            pltpu.VMEM((RB, K), jnp.int32),
            pltpu.VMEM((RB, K, D), jnp.float32),
            pltpu.VMEM_SHARED((NACC, D), jnp.float32),
            pltpu.SemaphoreType.DMA((RB,)),
            pltpu.SemaphoreType.DMA((RB,)),
            pltpu.SemaphoreType.DMA((NB,)),
            pltpu.SemaphoreType.DMA((RB,)),
        ],
    )


def _add_body(a_ref, b_ref, o_ref):
    o_ref[...] = a_ref[...] + b_ref[...]


_BM = 2000


def _add_partials(part):
    return pl.pallas_call(
        _add_body,
        grid=(N // _BM,),
        in_specs=[
            pl.BlockSpec((_BM, D), lambda i: (i, 0)),
            pl.BlockSpec((_BM, D), lambda i: (i + N // _BM, 0)),
        ],
        out_specs=pl.BlockSpec((_BM, D), lambda i: (i, 0)),
        out_shape=jax.ShapeDtypeStruct((N, D), jnp.float32),
    )(part, part)


@jax.jit
def kernel(edge_index, edge_weight, annotations):
    part = _sc_call()(edge_index.reshape(-1), edge_weight, annotations)
    return _add_partials(part)


# speedup vs baseline: 1.0084x; 1.0084x over previous
import functools

import jax
import jax.numpy as jnp
from jax import lax
from jax.experimental import pallas as pl
from jax.experimental.pallas import tpu as pltpu
from jax.experimental.pallas import tpu_sc as plsc

N = 10000
D = 128
E = 320000

NC = 2
NS = 16
NW = NC * NS
L = 16

K = 80
NB = 4
RB = 4
EPW = E // NW
NCHUNK = EPW // K
NACC = 10112
ZPT = NACC // NS
DPT = 624


def _sc_body(eidx_hbm, w_hbm, ann_hbm, part_hbm,
             col_v, row_v, w_v, srow_v, rows_v, acc, gsem, isem, ssem):
    cid = lax.axis_index("c")
    sid = lax.axis_index("s")
    wid = sid * NC + cid
    ebase = wid * EPW

    def idx_descs(c, q):
        base = pl.multiple_of(ebase + c * K, K)
        return (
            pltpu.make_async_copy(eidx_hbm.at[pl.ds(base, K)], row_v.at[q],
                                  isem.at[q]),
            pltpu.make_async_copy(eidx_hbm.at[pl.ds(E + base, K)],
                                  col_v.at[q], isem.at[q]),
            pltpu.make_async_copy(w_hbm.at[pl.ds(base, K)], w_v.at[q],
                                  isem.at[q]),
        )

    def gather_desc(q, p):
        return pltpu.make_async_copy(ann_hbm.at[col_v.at[q]], rows_v.at[p],
                                     gsem.at[p])

    def scatter_start(p):
        pltpu.async_copy(rows_v.at[p], acc.at[srow_v.at[p]], ssem.at[p],
                         add=True)

    def scatter_wait(p):
        pltpu.make_async_copy(rows_v.at[p], acc.at[srow_v.at[p]],
                              ssem.at[p]).wait()

    def mul_chunk(q, p):
        def mul_body(g, _):
            wv = w_v[q, pl.ds(g * L, L)]
            for l in range(L):
                e = g * L + l
                ws = jnp.full((L,), wv[l], jnp.float32)
                for j in range(D // L):
                    rows_v[p, e, pl.ds(j * L, L)] = (
                        rows_v[p, e, pl.ds(j * L, L)] * ws)
            return 0
        lax.fori_loop(0, K // L, mul_body, 0)
        for j in range(K // L):
            srow_v[p, pl.ds(j * L, L)] = row_v[q, pl.ds(j * L, L)]

    for c in range(NB):
        for d in idx_descs(c, c):
            d.start()

    def zero_rows(e, _):
        for j in range(D // L):
            rows_v[0, e, pl.ds(j * L, L)] = jnp.zeros((L,), jnp.float32)
        return 0
    lax.fori_loop(0, K, zero_rows, 0)
    zbase = sid * ZPT
    for i in range(ZPT // K):
        pltpu.sync_copy(rows_v.at[0, pl.ds(0, K)],
                        acc.at[pl.ds(zbase + i * K, K)])
    pltpu.sync_copy(rows_v.at[0, pl.ds(0, ZPT - (ZPT // K) * K)],
                    acc.at[pl.ds(zbase + (ZPT // K) * K,
                                 ZPT - (ZPT // K) * K)])
    plsc.subcore_barrier()

    for c in range(2):
        for d in idx_descs(c, c):
            d.wait()
        gather_desc(c, c).start()

    def chunk_body(c4, _):
        for k in range(NB):
            kn2 = (k + 2) % NB
            c = c4 * NB + k
            gather_desc(k, k).wait()

            @pl.when(c >= 2)
            def _():
                scatter_wait(kn2)

            @pl.when(c < NCHUNK - 2)
            def _():
                for d in idx_descs(c + 2, kn2):
                    d.wait()
                gather_desc(kn2, kn2).start()

            mul_chunk(k, k)
            scatter_start(k)

            @pl.when(c < NCHUNK - NB)
            def _():
                for d in idx_descs(c + NB, k):
                    d.start()
        return 0
    lax.fori_loop(0, (NCHUNK - 1) // NB, chunk_body, 0)

    kf = (NCHUNK - 1) % NB
    gather_desc(kf, kf).wait()
    scatter_wait((NCHUNK - 3) % RB)
    mul_chunk(kf, kf)
    scatter_start(kf)
    scatter_wait((NCHUNK - 2) % RB)
    scatter_wait(kf)
    plsc.subcore_barrier()

    dbase = DPT * sid + 8 * jnp.minimum(sid, 2)
    pltpu.sync_copy(acc.at[pl.ds(dbase, DPT)],
                    part_hbm.at[pl.ds(cid * N + dbase, DPT)])

    @pl.when(sid < 2)
    def _():
        pltpu.sync_copy(acc.at[pl.ds(dbase + DPT, 8)],
                        part_hbm.at[pl.ds(cid * N + dbase + DPT, 8)])


@functools.cache
def _sc_call():
    return pl.kernel(
        _sc_body,
        out_type=jax.ShapeDtypeStruct((NC * N, D), jnp.float32),
        mesh=plsc.VectorSubcoreMesh(core_axis_name="c", subcore_axis_name="s",
                                    num_cores=NC, num_subcores=NS),
        scratch_types=[
            pltpu.VMEM((NB, K), jnp.int32),
            pltpu.VMEM((NB, K), jnp.int32),
            pltpu.VMEM((NB, K), jnp.float32),
            pltpu.VMEM((RB, K), jnp.int32),
            pltpu.VMEM((RB, K, D), jnp.float32),
            pltpu.VMEM_SHARED((NACC, D), jnp.float32),
            pltpu.SemaphoreType.DMA((RB,)),
            pltpu.SemaphoreType.DMA((NB,)),
            pltpu.SemaphoreType.DMA((RB,)),
        ],
    )


def _add_body(a_ref, b_ref, o_ref):
    o_ref[...] = a_ref[...] + b_ref[...]


_BM = 2000


def _add_partials(part):
    return pl.pallas_call(
        _add_body,
        grid=(N // _BM,),
        in_specs=[
            pl.BlockSpec((_BM, D), lambda i: (i, 0)),
            pl.BlockSpec((_BM, D), lambda i: (i + N // _BM, 0)),
        ],
        out_specs=pl.BlockSpec((_BM, D), lambda i: (i, 0)),
        out_shape=jax.ShapeDtypeStruct((N, D), jnp.float32),
    )(part, part)


@jax.jit
def kernel(edge_index, edge_weight, annotations):
    part = _sc_call()(edge_index.reshape(-1), edge_weight, annotations)
    return _add_partials(part)
